# baseline (speedup 1.0000x reference)
import numpy as np

import jax
import jax.numpy as jnp
from jax.experimental import pallas as pl
from jax.experimental.pallas import tpu as pltpu


def _fused_cnn(x, w1, b1, w2, b2, fw1, fb1, fw2, fb2, *, tb):
    C, H, W, B = x.shape
    C1 = w1.shape[0]
    C2 = w2.shape[0]
    OH1 = (H - 8) // 4 + 1
    OW1 = (W - 8) // 4 + 1
    OH2 = (OH1 - 4) // 2 + 1
    OW2 = (OW1 - 4) // 2 + 1
    HID = fw1.shape[0]
    NP = fw2.shape[0]
    K1 = C * 64

    def body(xt_ref, w1_ref, b1_ref, w2_ref, b2_ref, fw1_ref,
             fb1_ref, fw2_ref, fb2_ref, o_ref, h1_ref, h2_ref):
        w1v = w1_ref[...]
        b1v = b1_ref[...]

        G1 = 4 if OW1 % 4 == 0 else 1
        for oh in range(OH1):
            for ow0 in range(0, OW1, G1):
                parts = [
                    xt_ref[:, pl.ds(4 * oh, 8), pl.ds(4 * (ow0 + j), 8), :]
                    .reshape(K1, tb) for j in range(G1)]
                rhs = parts[0] if G1 == 1 else jnp.concatenate(parts, axis=1)
                acc = jnp.dot(w1v, rhs, preferred_element_type=jnp.float32)
                acc = jnp.maximum(acc + b1v, 0.0).astype(jnp.bfloat16)
                for j in range(G1):
                    h1_ref[oh, ow0 + j, :, :] = acc[:, j * tb:(j + 1) * tb]

        w2v = w2_ref[...]
        b2v = b2_ref[...]

        for oh2 in range(OH2):
            for ow0 in range(0, OW2, 4):
                g = min(4, OW2 - ow0)
                parts = [
                    h1_ref[pl.ds(2 * oh2, 4), pl.ds(2 * (ow0 + j), 4), :, :]
                    .reshape(16 * C1, tb) for j in range(g)]
                rhs = parts[0] if g == 1 else jnp.concatenate(parts, axis=1)
                acc = jnp.dot(w2v, rhs, preferred_element_type=jnp.float32)
                acc = jnp.maximum(acc + b2v, 0.0).astype(jnp.bfloat16)
                for j in range(g):
                    h2_ref[oh2, ow0 + j, :, :] = acc[:, j * tb:(j + 1) * tb]

        flat = h2_ref[...].reshape(OH2 * OW2 * C2, tb)
        h = jnp.dot(fw1_ref[...], flat, preferred_element_type=jnp.float32)
        h = jnp.maximum(h + fb1_ref[...], 0.0).astype(jnp.bfloat16)
        o = jnp.dot(fw2_ref[...], h, preferred_element_type=jnp.float32)
        o_ref[0] = o + fb2_ref[...]

    def whole(a):
        return pl.BlockSpec(a.shape, lambda i: (0,) * a.ndim)

    return pl.pallas_call(
        body,
        grid=(B // tb,),
        in_specs=[
            pl.BlockSpec((C, H, W, tb), lambda i: (0, 0, 0, i)),
            whole(w1), whole(b1), whole(w2), whole(b2),
            whole(fw1), whole(fb1), whole(fw2), whole(fb2),
        ],
        out_specs=pl.BlockSpec((1, NP, tb), lambda i: (i, 0, 0)),
        out_shape=jax.ShapeDtypeStruct((B // tb, NP, tb), jnp.float32),
        scratch_shapes=[
            pltpu.VMEM((OH1, OW1, C1, tb), jnp.bfloat16),
            pltpu.VMEM((OH2, OW2, C2, tb), jnp.bfloat16),
        ],
        compiler_params=pltpu.CompilerParams(
            dimension_semantics=("parallel",),
            vmem_limit_bytes=60 * 1024 * 1024,
            allow_input_fusion=(True,) + (False,) * 8,
        ),
    )(x, w1, b1, w2, b2, fw1, fb1, fw2, fb2)


def kernel(x, c1_w, c1_b, c2_w, c2_b, fc1_w, fc1_b, fc2_w, fc2_b):
    B, C, H, W = x.shape
    C1 = c1_w.shape[0]
    C2 = c2_w.shape[0]
    OH1 = (H - 8) // 4 + 1
    OH2 = (OH1 - 4) // 2 + 1
    OW2 = OH2
    tb = 128 if B % 128 == 0 else B

    idx2 = np.array([c * 16 + kh * 4 + kw
                     for kh in range(4) for kw in range(4)
                     for c in range(C1)])
    w2 = c2_w[:, idx2].astype(jnp.bfloat16)

    idxf = np.array([c2 * (OH2 * OW2) + oh2 * OW2 + ow2
                     for oh2 in range(OH2) for ow2 in range(OW2)
                     for c2 in range(C2)])
    fw1 = fc1_w[idxf, :].T.astype(jnp.bfloat16)
    fb1 = fc1_b.reshape(-1, 1).astype(jnp.float32)
    fw2 = fc2_w.T.astype(jnp.bfloat16)
    fb2 = fc2_b.reshape(-1, 1).astype(jnp.float32)

    nch = 4 if B % (4 * tb) == 0 else 1
    chunk = B // nch
    w1 = c1_w.astype(jnp.float32)
    b1 = c1_b.astype(jnp.float32)
    b2 = c2_b.astype(jnp.float32)
    outs = []
    for k in range(nch):
        xs = jax.lax.slice_in_dim(x, k * chunk, (k + 1) * chunk, axis=0)
        xtk = jnp.transpose(xs, (1, 2, 3, 0)).astype(jnp.float32)
        outs.append(_fused_cnn(xtk, w1, b1, w2, b2, fw1, fb1, fw2, fb2,
                               tb=tb))
    out = jnp.concatenate(outs, axis=0) if nch > 1 else outs[0]
    return jnp.swapaxes(out, 1, 2).reshape(B, -1)[:, :18]


# speedup vs baseline: 1.0572x; 1.0572x over previous
import numpy as np

import jax
import jax.numpy as jnp
from jax.experimental import pallas as pl
from jax.experimental.pallas import tpu as pltpu


def _fused_cnn(x, w1, b1, w2, b2, fw1, fb1, fw2, fb2, *, tb):
    C, H, W, B = x.shape
    C1 = w1.shape[0]
    C2 = w2.shape[0]
    OH1 = (H - 8) // 4 + 1
    OW1 = (W - 8) // 4 + 1
    OH2 = (OH1 - 4) // 2 + 1
    OW2 = (OW1 - 4) // 2 + 1
    HID = fw1.shape[0]
    NP = fw2.shape[0]
    K1 = C * 64

    def body(xt_ref, w1_ref, b1_ref, w2_ref, b2_ref, fw1_ref,
             fb1_ref, fw2_ref, fb2_ref, o_ref, xf_ref, h1_ref, h2_ref):
        xf_ref[...] = xt_ref[...].astype(jnp.float32)
        w1v = w1_ref[...]
        b1v = b1_ref[...]

        G1 = 4 if OW1 % 4 == 0 else 1
        for oh in range(OH1):
            for ow0 in range(0, OW1, G1):
                parts = [
                    xf_ref[:, pl.ds(4 * oh, 8), pl.ds(4 * (ow0 + j), 8), :]
                    .reshape(K1, tb) for j in range(G1)]
                rhs = parts[0] if G1 == 1 else jnp.concatenate(parts, axis=1)
                acc = jnp.dot(w1v, rhs, preferred_element_type=jnp.float32)
                acc = jnp.maximum(acc + b1v, 0.0).astype(jnp.bfloat16)
                for j in range(G1):
                    h1_ref[oh, ow0 + j, :, :] = acc[:, j * tb:(j + 1) * tb]

        w2v = w2_ref[...]
        b2v = b2_ref[...]

        for oh2 in range(OH2):
            for ow0 in range(0, OW2, 4):
                g = min(4, OW2 - ow0)
                parts = [
                    h1_ref[pl.ds(2 * oh2, 4), pl.ds(2 * (ow0 + j), 4), :, :]
                    .reshape(16 * C1, tb) for j in range(g)]
                rhs = parts[0] if g == 1 else jnp.concatenate(parts, axis=1)
                acc = jnp.dot(w2v, rhs, preferred_element_type=jnp.float32)
                acc = jnp.maximum(acc + b2v, 0.0).astype(jnp.bfloat16)
                for j in range(g):
                    h2_ref[oh2, ow0 + j, :, :] = acc[:, j * tb:(j + 1) * tb]

        flat = h2_ref[...].reshape(OH2 * OW2 * C2, tb)
        h = jnp.dot(fw1_ref[...], flat, preferred_element_type=jnp.float32)
        h = jnp.maximum(h + fb1_ref[...], 0.0).astype(jnp.bfloat16)
        o = jnp.dot(fw2_ref[...], h, preferred_element_type=jnp.float32)
        o_ref[0] = o + fb2_ref[...]

    def whole(a):
        return pl.BlockSpec(a.shape, lambda i: (0,) * a.ndim)

    return pl.pallas_call(
        body,
        grid=(B // tb,),
        in_specs=[
            pl.BlockSpec((C, H, W, tb), lambda i: (0, 0, 0, i)),
            whole(w1), whole(b1), whole(w2), whole(b2),
            whole(fw1), whole(fb1), whole(fw2), whole(fb2),
        ],
        out_specs=pl.BlockSpec((1, NP, tb), lambda i: (i, 0, 0)),
        out_shape=jax.ShapeDtypeStruct((B // tb, NP, tb), jnp.float32),
        scratch_shapes=[
            pltpu.VMEM((C, H, W, tb), jnp.float32),
            pltpu.VMEM((OH1, OW1, C1, tb), jnp.bfloat16),
            pltpu.VMEM((OH2, OW2, C2, tb), jnp.bfloat16),
        ],
        compiler_params=pltpu.CompilerParams(
            dimension_semantics=("parallel",),
            vmem_limit_bytes=60 * 1024 * 1024,
            allow_input_fusion=(True,) + (False,) * 8,
        ),
    )(x, w1, b1, w2, b2, fw1, fb1, fw2, fb2)


def kernel(x, c1_w, c1_b, c2_w, c2_b, fc1_w, fc1_b, fc2_w, fc2_b):
    B, C, H, W = x.shape
    C1 = c1_w.shape[0]
    C2 = c2_w.shape[0]
    OH1 = (H - 8) // 4 + 1
    OH2 = (OH1 - 4) // 2 + 1
    OW2 = OH2
    tb = 128 if B % 128 == 0 else B

    idx2 = np.array([c * 16 + kh * 4 + kw
                     for kh in range(4) for kw in range(4)
                     for c in range(C1)])
    w2 = c2_w[:, idx2].astype(jnp.bfloat16)

    idxf = np.array([c2 * (OH2 * OW2) + oh2 * OW2 + ow2
                     for oh2 in range(OH2) for ow2 in range(OW2)
                     for c2 in range(C2)])
    fw1 = fc1_w[idxf, :].T.astype(jnp.bfloat16)
    fb1 = fc1_b.reshape(-1, 1).astype(jnp.float32)
    fw2 = fc2_w.T.astype(jnp.bfloat16)
    fb2 = fc2_b.reshape(-1, 1).astype(jnp.float32)

    nch = 4 if B % (4 * tb) == 0 else 1
    chunk = B // nch
    w1 = c1_w.astype(jnp.float32)
    b1 = c1_b.astype(jnp.float32)
    b2 = c2_b.astype(jnp.float32)
    outs = []
    for k in range(nch):
        xs = jax.lax.slice_in_dim(x, k * chunk, (k + 1) * chunk, axis=0)
        xtk = jnp.transpose(xs, (1, 2, 3, 0)).astype(jnp.bfloat16)
        outs.append(_fused_cnn(xtk, w1, b1, w2, b2, fw1, fb1, fw2, fb2,
                               tb=tb))
    out = jnp.concatenate(outs, axis=0) if nch > 1 else outs[0]
    return jnp.swapaxes(out, 1, 2).reshape(B, -1)[:, :18]
